# baseline (speedup 1.0000x reference)
import functools

import jax
import jax.numpy as jnp
from jax import lax
from jax.experimental import pallas as pl
from jax.experimental.pallas import tpu as pltpu
from jax.experimental.pallas import tpu_sc as plsc

_ROWS_PER_BLOCK = 8
_INT_MIN = -(2 ** 31)
_NB_FINE = 1 << 16
_NB_COARSE = 1 << 12
_NW = 32


def _dec(enc):
    bits = jnp.where(enc >= 0, enc, jnp.int32(_INT_MIN) - enc)
    return jax.lax.bitcast_convert_type(bits, jnp.float32)


def _sampler_block(temp_ref, tp_ref, tk_ref, thr_ref, logits_ref, noise_ref,
                   out_ref):
    temp = temp_ref[...]
    greedy = temp <= 1e-10
    safe = jnp.where(greedy, 1.0, temp)
    l = logits_ref[...] / safe
    m = jnp.max(l, axis=1, keepdims=True)
    kf = tk_ref[...].astype(jnp.float32)

    thr = thr_ref[...] / safe
    surv = (kf <= 0.0) | (l >= thr)
    e_m = jnp.where(surv, jnp.exp(l - m), 0.0)
    zk = jnp.sum(e_m, axis=1, keepdims=True)
    q = e_m / zk
    p = tp_ref[...]

    def topp_bit(i, t):
        cand = t + (jnp.int32(1) << (30 - i))
        mass = jnp.sum(jnp.where(q >= _dec(cand), q, 0.0),
                       axis=1, keepdims=True)
        return jnp.where(mass > p, cand, t)

    t2 = jax.lax.fori_loop(1, 31, topp_bit, jnp.zeros(temp.shape, jnp.int32))

    mxq = jnp.max(q, axis=1, keepdims=True)
    nmx = jnp.sum(jnp.where(q == mxq, 1.0, 0.0), axis=1, keepdims=True)
    s2 = jnp.where(nmx >= 2.0, mxq,
                   jnp.max(jnp.where(q < mxq, q, 0.0), axis=1, keepdims=True))
    c_val = jnp.minimum(_dec(t2), s2)

    gt = q > c_val
    tie = q == c_val
    s_gt = jnp.sum(jnp.where(gt, q, 0.0), axis=1, keepdims=True)
    cnt_gt = jnp.sum(jnp.where(gt, 1.0, 0.0), axis=1, keepdims=True)
    cnt_c = jnp.sum(jnp.where(tie, 1.0, 0.0), axis=1, keepdims=True)
    c_safe = jnp.maximum(c_val, 1e-30)
    n_cross = jnp.where(
        s_gt <= p,
        jnp.floor(jnp.minimum((p - s_gt) / c_safe, 1e9)) + 1.0,
        0.0)
    n_forced = jnp.maximum(2.0 - cnt_gt, 0.0)
    n_c = jnp.minimum(jnp.maximum(n_cross, n_forced), cnt_c)
    iota = jax.lax.broadcasted_iota(jnp.int32, q.shape, 1)

    big = jnp.int32(2 ** 31 - 1)

    def tieidx_bit(i, x):
        cand = x + (jnp.int32(1) << (16 - i))
        cnt = jnp.sum(jnp.where(tie & (iota < cand), 1.0, 0.0),
                      axis=1, keepdims=True)
        return jnp.where(cnt < n_c, cand, x)

    need_search = (n_c >= 1.0) & (n_c < cnt_c) & (cnt_c > 1.0)
    xh = jax.lax.cond(
        jnp.any(need_search),
        lambda: jax.lax.fori_loop(0, 17, tieidx_bit,
                                  jnp.zeros(temp.shape, jnp.int32)),
        lambda: jnp.full(temp.shape, big, jnp.int32))
    kept = gt | (tie & (iota <= xh) & (n_c >= 1.0))

    race = jnp.where(kept, q, 0.0) / noise_ref[...]
    val = jnp.where(greedy, l, race)
    vmx = jnp.max(val, axis=1, keepdims=True)
    out_ref[...] = jnp.min(jnp.where(val == vmx, iota, big),
                           axis=1, keepdims=True)


def _make_sc_kth(b, v, ch):
    nch = v // ch
    nvec = ch // 16
    rpw = b // _NW
    mesh = plsc.VectorSubcoreMesh(core_axis_name="c", subcore_axis_name="s")

    @functools.partial(
        pl.kernel,
        mesh=mesh,
        compiler_params=pltpu.CompilerParams(needs_layout_passes=False),
        out_type=jax.ShapeDtypeStruct((b * 16,), jnp.float32),
        scratch_types=[
            pltpu.VMEM((ch,), jnp.float32),
            pltpu.VMEM((ch,), jnp.float32),
            pltpu.VMEM((_NB_FINE,), jnp.int32),
            pltpu.VMEM((_NB_COARSE,), jnp.int32),
            pltpu.VMEM((16,), jnp.int32),
            pltpu.VMEM((16,), jnp.int32),
            pltpu.VMEM((16,), jnp.float32),
            pltpu.SemaphoreType.DMA,
            pltpu.SemaphoreType.DMA,
        ],
    )
    def sc_kth(logits_hbm, ks_hbm, out_hbm, buf0, buf1, fine, coarse, acc,
               kbuf, obuf, sem0, sem1):
        wid = lax.axis_index("s") * 2 + lax.axis_index("c")
        iota16 = lax.broadcasted_iota(jnp.int32, (16,), 0)
        ones16 = jnp.ones((16,), jnp.int32)
        zeros16 = jnp.zeros((16,), jnp.int32)
        bufs = (buf0, buf1)
        sems = (sem0, sem1)

        def encode(x):
            bi = lax.bitcast_convert_type(x, jnp.int32)
            s = lax.shift_right_arithmetic(bi, 31)
            return bi ^ (s | jnp.int32(_INT_MIN))

        def zero_hists():
            def zf(i, _):
                fine[pl.ds(i * 16, 16)] = zeros16
                return 0
            lax.fori_loop(0, _NB_FINE // 16, zf, 0)

            def zc(i, _):
                coarse[pl.ds(i * 16, 16)] = zeros16
                return 0
            lax.fori_loop(0, _NB_COARSE // 16, zc, 0)

        def stream(row, body):
            h = pltpu.async_copy(logits_hbm.at[pl.ds(row * v, ch)],
                                 bufs[0], sems[0])
            for c in range(nch):
                if c + 1 < nch:
                    h_next = pltpu.async_copy(
                        logits_hbm.at[pl.ds(row * v + (c + 1) * ch, ch)],
                        bufs[(c + 1) % 2], sems[(c + 1) % 2])
                h.wait()
                buf = bufs[c % 2]

                def ib(i, _):
                    body(buf[pl.ds(i * 16, 16)])
                    return 0
                lax.fori_loop(0, nvec, ib, 0)
                if c + 1 < nch:
                    h = h_next

        def vsum(vec):
            acc[...] = zeros16
            plsc.addupdate_scatter(acc, [zeros16], vec)
            return acc[...][0]

        def lane_search(vec, s_above, k_needed):
            lane = jnp.int32(0)
            for bit in (8, 4, 2, 1):
                cand = lane + jnp.int32(bit)
                s = vsum(jnp.where(iota16 >= cand, vec, 0))
                lane = jnp.where(s_above + s >= k_needed, cand, lane)
            sat = s_above + vsum(jnp.where(iota16 >= lane, vec, 0))
            cnt = vsum(jnp.where(iota16 == lane, vec, 0))
            return lane, sat, cnt

        def find_cross(k_needed):
            acc[...] = zeros16

            def cs(j, carry):
                s_run, base_best, s_above_best = carry
                base = _NB_COARSE - 16 * (j + 1)
                cvec = coarse[pl.ds(base, 16)]
                plsc.addupdate_scatter(acc, [zeros16], cvec)
                s_after = acc[...][0]
                crossing = (s_run < k_needed) & (s_after >= k_needed)
                return (s_after,
                        jnp.where(crossing, jnp.int32(base), base_best),
                        jnp.where(crossing, s_run, s_above_best))

            _, base_c, s_above2 = lax.fori_loop(
                0, _NB_COARSE // 16, cs,
                (jnp.int32(0), jnp.int32(0), jnp.int32(0)))
            cvec = coarse[pl.ds(base_c, 16)]
            lane2, sat2, cnt2 = lane_search(cvec, s_above2, k_needed)
            bc = base_c + lane2
            fvec = fine[pl.ds(bc * 16, 16)]
            lane3, satf, cnt = lane_search(fvec, sat2 - cnt2, k_needed)
            return bc * 16 + lane3, satf, cnt

        for j in range(rpw):
            row = wid * rpw + j
            pltpu.sync_copy(ks_hbm.at[pl.ds(row * 16, 16)], kbuf)
            k_needed = kbuf[...][0]

            zero_hists()

            def p1(x):
                e = encode(x)
                plsc.addupdate_scatter(
                    fine, [lax.shift_right_logical(e, 16)], ones16)
                plsc.addupdate_scatter(
                    coarse, [lax.shift_right_logical(e, 20)], ones16)
            stream(row, p1)
            b1, s1, c1 = find_cross(k_needed)
            k2 = k_needed - (s1 - c1)

            zero_hists()

            def p2(x):
                e = encode(x)
                msk = lax.shift_right_logical(e, 16) == b1
                lo = e & jnp.int32(0xFFFF)
                plsc.addupdate_scatter(fine, [lo], ones16, mask=msk)
                plsc.addupdate_scatter(
                    coarse, [lax.shift_right_logical(lo, 4)], ones16,
                    mask=msk)
            stream(row, p2)
            b2, _, _ = find_cross(k2)

            encv = jnp.broadcast_to((b1 << 16) | b2, (16,))
            bits = jnp.where(encv >= 0, encv ^ jnp.int32(-1),
                             encv ^ jnp.int32(_INT_MIN))
            obuf[...] = lax.bitcast_convert_type(bits, jnp.float32)
            pltpu.sync_copy(obuf, out_hbm.at[pl.ds(row * 16, 16)])

    return sc_kth


_sc_cache = {}


def _sc_kth_call(logits, ks):
    b, v = logits.shape
    ch = 20000 if v % 20000 == 0 else v
    key = (b, v, ch)
    if key not in _sc_cache:
        _sc_cache[key] = _make_sc_kth(b, v, ch)
    out = _sc_cache[key](logits.reshape(b * v), ks.reshape(b * 16))
    return out.reshape(b, 16)


_noise_cache = {}


def _padded_noise(shape, vpad):
    key = (shape, vpad)
    if key not in _noise_cache:
        n = jnp.maximum(
            jax.random.exponential(jax.random.key(42), shape, jnp.float32),
            1e-10)
        n = jnp.pad(n, ((0, 0), (0, vpad - shape[1])), constant_values=1.0)
        _noise_cache[key] = jax.block_until_ready(n)
    return _noise_cache[key]


def kernel(logits, temperatures, top_ps, top_ks):
    logits = logits.astype(jnp.float32)
    b, v = logits.shape
    vp = ((v + 127) // 128) * 128
    lp = jnp.pad(logits, ((0, 0), (0, vp - v)), constant_values=-jnp.inf)
    noise = _padded_noise((b, v), vp)
    t2d = temperatures.astype(jnp.float32).reshape(b, 1)
    p2d = top_ps.astype(jnp.float32).reshape(b, 1)
    k2d = jnp.minimum(top_ks, v).astype(jnp.int32).reshape(b, 1)

    ks_sc = jnp.broadcast_to(jnp.maximum(k2d, 1), (b, 16)).astype(jnp.int32)
    thr = _sc_kth_call(logits, ks_sc)[:, :1]

    r = min(_ROWS_PER_BLOCK, b)
    out = pl.pallas_call(
        _sampler_block,
        grid=(b // r,),
        in_specs=[
            pl.BlockSpec((r, 1), lambda i: (i, 0)),
            pl.BlockSpec((r, 1), lambda i: (i, 0)),
            pl.BlockSpec((r, 1), lambda i: (i, 0)),
            pl.BlockSpec((r, 1), lambda i: (i, 0)),
            pl.BlockSpec((r, vp), lambda i: (i, 0)),
            pl.BlockSpec((r, vp), lambda i: (i, 0)),
        ],
        out_specs=pl.BlockSpec((r, 1), lambda i: (i, 0)),
        out_shape=jax.ShapeDtypeStruct((b, 1), jnp.int32),
    )(t2d, p2d, k2d, thr, lp, noise)
    return out.reshape(b)


# speedup vs baseline: 1.1289x; 1.1289x over previous
import functools

import jax
import jax.numpy as jnp
from jax import lax
from jax.experimental import pallas as pl
from jax.experimental.pallas import tpu as pltpu
from jax.experimental.pallas import tpu_sc as plsc

_ROWS_PER_BLOCK = 16
_INT_MIN = -(2 ** 31)
_NB_FINE = 1 << 16
_NB_COARSE = 1 << 12
_NW = 32


def _dec(enc):
    bits = jnp.where(enc >= 0, enc, jnp.int32(_INT_MIN) - enc)
    return jax.lax.bitcast_convert_type(bits, jnp.float32)


def _sampler_block(temp_ref, tp_ref, tk_ref, thr_ref, logits_ref, noise_ref,
                   out_ref):
    temp = temp_ref[...]
    greedy = temp <= 1e-10
    safe = jnp.where(greedy, 1.0, temp)
    l = logits_ref[...] / safe
    m = jnp.max(l, axis=1, keepdims=True)
    kf = tk_ref[...].astype(jnp.float32)

    thr = thr_ref[...] / safe
    surv = (kf <= 0.0) | (l >= thr)
    e_m = jnp.where(surv, jnp.exp(l - m), 0.0)
    zk = jnp.sum(e_m, axis=1, keepdims=True)
    q = e_m / zk
    p = tp_ref[...]

    def topp_bit(i, t):
        cand = t + (jnp.int32(1) << (30 - i))
        mass = jnp.sum(jnp.where(q >= _dec(cand), q, 0.0),
                       axis=1, keepdims=True)
        return jnp.where(mass > p, cand, t)

    t2 = jax.lax.fori_loop(1, 31, topp_bit, jnp.zeros(temp.shape, jnp.int32))

    mxq = jnp.max(q, axis=1, keepdims=True)
    nmx = jnp.sum(jnp.where(q == mxq, 1.0, 0.0), axis=1, keepdims=True)
    s2 = jnp.where(nmx >= 2.0, mxq,
                   jnp.max(jnp.where(q < mxq, q, 0.0), axis=1, keepdims=True))
    c_val = jnp.minimum(_dec(t2), s2)

    gt = q > c_val
    tie = q == c_val
    s_gt = jnp.sum(jnp.where(gt, q, 0.0), axis=1, keepdims=True)
    cnt_gt = jnp.sum(jnp.where(gt, 1.0, 0.0), axis=1, keepdims=True)
    cnt_c = jnp.sum(jnp.where(tie, 1.0, 0.0), axis=1, keepdims=True)
    c_safe = jnp.maximum(c_val, 1e-30)
    n_cross = jnp.where(
        s_gt <= p,
        jnp.floor(jnp.minimum((p - s_gt) / c_safe, 1e9)) + 1.0,
        0.0)
    n_forced = jnp.maximum(2.0 - cnt_gt, 0.0)
    n_c = jnp.minimum(jnp.maximum(n_cross, n_forced), cnt_c)
    iota = jax.lax.broadcasted_iota(jnp.int32, q.shape, 1)

    big = jnp.int32(2 ** 31 - 1)

    def tieidx_bit(i, x):
        cand = x + (jnp.int32(1) << (16 - i))
        cnt = jnp.sum(jnp.where(tie & (iota < cand), 1.0, 0.0),
                      axis=1, keepdims=True)
        return jnp.where(cnt < n_c, cand, x)

    need_search = (n_c >= 1.0) & (n_c < cnt_c) & (cnt_c > 1.0)
    xh = jax.lax.cond(
        jnp.any(need_search),
        lambda: jax.lax.fori_loop(0, 17, tieidx_bit,
                                  jnp.zeros(temp.shape, jnp.int32)),
        lambda: jnp.full(temp.shape, big, jnp.int32))
    kept = gt | (tie & (iota <= xh) & (n_c >= 1.0))

    race = jnp.where(kept, q, 0.0) / noise_ref[...]
    val = jnp.where(greedy, l, race)
    vmx = jnp.max(val, axis=1, keepdims=True)
    out_ref[...] = jnp.min(jnp.where(val == vmx, iota, big),
                           axis=1, keepdims=True)


def _make_sc_kth(b, v, ch):
    nch = v // ch
    nvec = ch // 16
    rpw = b // _NW
    mesh = plsc.VectorSubcoreMesh(core_axis_name="c", subcore_axis_name="s")

    @functools.partial(
        pl.kernel,
        mesh=mesh,
        compiler_params=pltpu.CompilerParams(needs_layout_passes=False),
        out_type=jax.ShapeDtypeStruct((b * 16,), jnp.float32),
        scratch_types=[
            pltpu.VMEM((ch,), jnp.float32),
            pltpu.VMEM((ch,), jnp.float32),
            pltpu.VMEM((_NB_FINE,), jnp.int32),
            pltpu.VMEM((_NB_COARSE,), jnp.int32),
            pltpu.VMEM((16,), jnp.int32),
            pltpu.VMEM((16,), jnp.int32),
            pltpu.VMEM((16,), jnp.float32),
            pltpu.SemaphoreType.DMA,
            pltpu.SemaphoreType.DMA,
        ],
    )
    def sc_kth(logits_hbm, ks_hbm, out_hbm, buf0, buf1, fine, coarse, acc,
               kbuf, obuf, sem0, sem1):
        wid = lax.axis_index("s") * 2 + lax.axis_index("c")
        iota16 = lax.broadcasted_iota(jnp.int32, (16,), 0)
        ones16 = jnp.ones((16,), jnp.int32)
        zeros16 = jnp.zeros((16,), jnp.int32)
        bufs = (buf0, buf1)
        sems = (sem0, sem1)

        def encode(x):
            bi = lax.bitcast_convert_type(x, jnp.int32)
            s = lax.shift_right_arithmetic(bi, 31)
            return bi ^ (s | jnp.int32(_INT_MIN))

        def zero_hists():
            def zf(i, _):
                fine[pl.ds(i * 16, 16)] = zeros16
                return 0
            lax.fori_loop(0, _NB_FINE // 16, zf, 0)

            def zc(i, _):
                coarse[pl.ds(i * 16, 16)] = zeros16
                return 0
            lax.fori_loop(0, _NB_COARSE // 16, zc, 0)

        def stream(row, body):
            h = pltpu.async_copy(logits_hbm.at[pl.ds(row * v, ch)],
                                 bufs[0], sems[0])
            for c in range(nch):
                if c + 1 < nch:
                    h_next = pltpu.async_copy(
                        logits_hbm.at[pl.ds(row * v + (c + 1) * ch, ch)],
                        bufs[(c + 1) % 2], sems[(c + 1) % 2])
                h.wait()
                buf = bufs[c % 2]

                def ib(i, _):
                    body(buf[pl.ds(i * 16, 16)])
                    return 0
                lax.fori_loop(0, nvec, ib, 0)
                if c + 1 < nch:
                    h = h_next

        def vsum(vec):
            acc[...] = zeros16
            plsc.addupdate_scatter(acc, [zeros16], vec)
            return acc[...][0]

        def lane_search(vec, s_above, k_needed):
            lane = jnp.int32(0)
            for bit in (8, 4, 2, 1):
                cand = lane + jnp.int32(bit)
                s = vsum(jnp.where(iota16 >= cand, vec, 0))
                lane = jnp.where(s_above + s >= k_needed, cand, lane)
            sat = s_above + vsum(jnp.where(iota16 >= lane, vec, 0))
            cnt = vsum(jnp.where(iota16 == lane, vec, 0))
            return lane, sat, cnt

        def find_cross(k_needed):
            acc[...] = zeros16

            def cs(j, carry):
                s_run, base_best, s_above_best = carry
                base = _NB_COARSE - 16 * (j + 1)
                cvec = coarse[pl.ds(base, 16)]
                plsc.addupdate_scatter(acc, [zeros16], cvec)
                s_after = acc[...][0]
                crossing = (s_run < k_needed) & (s_after >= k_needed)
                return (s_after,
                        jnp.where(crossing, jnp.int32(base), base_best),
                        jnp.where(crossing, s_run, s_above_best))

            _, base_c, s_above2 = lax.fori_loop(
                0, _NB_COARSE // 16, cs,
                (jnp.int32(0), jnp.int32(0), jnp.int32(0)))
            cvec = coarse[pl.ds(base_c, 16)]
            lane2, sat2, cnt2 = lane_search(cvec, s_above2, k_needed)
            bc = base_c + lane2
            fvec = fine[pl.ds(bc * 16, 16)]
            lane3, satf, cnt = lane_search(fvec, sat2 - cnt2, k_needed)
            return bc * 16 + lane3, satf, cnt

        for j in range(rpw):
            row = wid * rpw + j
            pltpu.sync_copy(ks_hbm.at[pl.ds(row * 16, 16)], kbuf)
            k_needed = kbuf[...][0]

            zero_hists()

            def p1(x):
                e = encode(x)
                plsc.addupdate_scatter(
                    fine, [lax.shift_right_logical(e, 16)], ones16)
                plsc.addupdate_scatter(
                    coarse, [lax.shift_right_logical(e, 20)], ones16)
            stream(row, p1)
            b1, s1, c1 = find_cross(k_needed)
            k2 = k_needed - (s1 - c1)

            zero_hists()

            def p2(x):
                e = encode(x)
                msk = lax.shift_right_logical(e, 16) == b1
                lo = e & jnp.int32(0xFFFF)
                plsc.addupdate_scatter(fine, [lo], ones16, mask=msk)
                plsc.addupdate_scatter(
                    coarse, [lax.shift_right_logical(lo, 4)], ones16,
                    mask=msk)
            stream(row, p2)
            b2, _, _ = find_cross(k2)

            encv = jnp.broadcast_to((b1 << 16) | b2, (16,))
            bits = jnp.where(encv >= 0, encv ^ jnp.int32(-1),
                             encv ^ jnp.int32(_INT_MIN))
            obuf[...] = lax.bitcast_convert_type(bits, jnp.float32)
            pltpu.sync_copy(obuf, out_hbm.at[pl.ds(row * 16, 16)])

    return sc_kth


_sc_cache = {}


def _sc_kth_call(logits, ks):
    b, v = logits.shape
    ch = 20000 if v % 20000 == 0 else v
    key = (b, v, ch)
    if key not in _sc_cache:
        _sc_cache[key] = _make_sc_kth(b, v, ch)
    out = _sc_cache[key](logits.reshape(b * v), ks.reshape(b * 16))
    return out.reshape(b, 16)


_noise_cache = {}


def _padded_noise(shape, vpad):
    key = (shape, vpad)
    if key not in _noise_cache:
        n = jnp.maximum(
            jax.random.exponential(jax.random.key(42), shape, jnp.float32),
            1e-10)
        n = jnp.pad(n, ((0, 0), (0, vpad - shape[1])), constant_values=1.0)
        _noise_cache[key] = jax.block_until_ready(n)
    return _noise_cache[key]


def kernel(logits, temperatures, top_ps, top_ks):
    logits = logits.astype(jnp.float32)
    b, v = logits.shape
    vp = ((v + 127) // 128) * 128
    lp = jnp.pad(logits, ((0, 0), (0, vp - v)), constant_values=-jnp.inf)
    noise = _padded_noise((b, v), vp)
    t2d = temperatures.astype(jnp.float32).reshape(b, 1)
    p2d = top_ps.astype(jnp.float32).reshape(b, 1)
    k2d = jnp.minimum(top_ks, v).astype(jnp.int32).reshape(b, 1)

    ks_sc = jnp.broadcast_to(jnp.maximum(k2d, 1), (b, 16)).astype(jnp.int32)
    thr = _sc_kth_call(logits, ks_sc)[:, :1]

    r = min(_ROWS_PER_BLOCK, b)
    out = pl.pallas_call(
        _sampler_block,
        grid=(b // r,),
        in_specs=[
            pl.BlockSpec((r, 1), lambda i: (i, 0)),
            pl.BlockSpec((r, 1), lambda i: (i, 0)),
            pl.BlockSpec((r, 1), lambda i: (i, 0)),
            pl.BlockSpec((r, 1), lambda i: (i, 0)),
            pl.BlockSpec((r, vp), lambda i: (i, 0)),
            pl.BlockSpec((r, vp), lambda i: (i, 0)),
        ],
        out_specs=pl.BlockSpec((r, 1), lambda i: (i, 0)),
        out_shape=jax.ShapeDtypeStruct((b, 1), jnp.int32),
    )(t2d, p2d, k2d, thr, lp, noise)
    return out.reshape(b)
